# baseline (speedup 1.0000x reference)
import functools

import jax
import jax.numpy as jnp
from jax import lax
from jax.experimental import pallas as pl
from jax.experimental.pallas import tpu as pltpu
from jax.experimental.pallas import tpu_sc as plsc

GAMMA = 12.0
EMBED_DIM = 64
LANES = 16
NUM_CORES = 2
NUM_SUBCORES = 16
NUM_WORKERS = NUM_CORES * NUM_SUBCORES
GATHER_CHUNK = 128


def _sc_body(b_per_w, n_chunks,
             head_idx_hbm, rel_idx_hbm, tail_idx_hbm, ent_hbm, rel_hbm,
             out_hbm,
             hidx_v, ridx_v, tidx_v, head_v, tail_v, relg_v, out_v, tscr,
             sem):
    wid = lax.axis_index("s") * NUM_CORES + lax.axis_index("c")
    base = wid * b_per_w
    chunk_base = wid * n_chunks

    pltpu.sync_copy(head_idx_hbm.at[pl.ds(chunk_base, n_chunks)], hidx_v)
    pltpu.sync_copy(rel_idx_hbm.at[pl.ds(chunk_base, n_chunks)], ridx_v)
    pltpu.sync_copy(tail_idx_hbm.at[pl.ds(chunk_base, n_chunks)], tidx_v)

    copies = []
    for c in range(n_chunks):
        rows = pl.ds(c * GATHER_CHUNK, GATHER_CHUNK)
        copies.append(pltpu.async_copy(
            ent_hbm.at[hidx_v.at[c]], head_v.at[rows], sem))
        copies.append(pltpu.async_copy(
            ent_hbm.at[tidx_v.at[c]], tail_v.at[rows], sem))
        copies.append(pltpu.async_copy(
            rel_hbm.at[ridx_v.at[c]], relg_v.at[rows], sem))
    for cp in copies:
        cp.wait()

    iota16 = lax.iota(jnp.int32, LANES)
    n_groups = b_per_w // LANES

    def group_body(g, carry):
        for j in range(LANES):
            row = g * LANES + j
            acc = jnp.zeros((LANES,), jnp.float32)
            for c in range(EMBED_DIM // LANES):
                sl = pl.ds(c * LANES, LANES)
                h = head_v[row, sl]
                t = tail_v[row, sl]
                r = relg_v[row, sl]
                gate = 1.0 / (1.0 + jnp.exp(-r))
                d = (h - t) * gate
                acc = acc + d * d
            tscr[j, :] = acc
        tot = jnp.zeros((LANES,), jnp.float32)
        for d in range(LANES):
            col = plsc.load_gather(
                tscr, [iota16, jnp.full((LANES,), d, jnp.int32)])
            tot = tot + col
        seed = plsc.bitcast(
            jnp.int32(0x5F3759DF) - (plsc.bitcast(tot, jnp.int32) >> 1),
            jnp.float32)
        y = seed
        for _ in range(3):
            y = y * (1.5 - 0.5 * tot * y * y)
        dist = jnp.where(tot > 0.0, tot * y, 0.0)
        out_v[pl.ds(g * LANES, LANES)] = GAMMA - dist
        return carry

    lax.fori_loop(0, n_groups, group_body, 0)
    pltpu.sync_copy(out_v, out_hbm.at[pl.ds(base, b_per_w)])


def kernel(head_idx, relation_idx, tail_idx, entity_table, relation_table):
    batch = head_idx.shape[0]
    b_per_w = batch // NUM_WORKERS
    n_chunks = b_per_w // GATHER_CHUNK

    mesh = plsc.VectorSubcoreMesh(core_axis_name="c", subcore_axis_name="s")
    sc_kernel = functools.partial(
        pl.kernel,
        out_type=jax.ShapeDtypeStruct((batch,), jnp.float32),
        mesh=mesh,
        compiler_params=pltpu.CompilerParams(
            needs_layout_passes=False, use_tc_tiling_on_sc=False),
        scratch_types=[
            pltpu.VMEM((n_chunks, GATHER_CHUNK), jnp.int32),
            pltpu.VMEM((n_chunks, GATHER_CHUNK), jnp.int32),
            pltpu.VMEM((n_chunks, GATHER_CHUNK), jnp.int32),
            pltpu.VMEM((b_per_w, EMBED_DIM), jnp.float32),
            pltpu.VMEM((b_per_w, EMBED_DIM), jnp.float32),
            pltpu.VMEM((b_per_w, EMBED_DIM), jnp.float32),
            pltpu.VMEM((b_per_w,), jnp.float32),
            pltpu.VMEM((LANES, LANES), jnp.float32),
            pltpu.SemaphoreType.DMA,
        ],
    )(functools.partial(_sc_body, b_per_w, n_chunks))

    total_chunks = batch // GATHER_CHUNK
    return sc_kernel(
        head_idx.reshape(total_chunks, GATHER_CHUNK),
        relation_idx.reshape(total_chunks, GATHER_CHUNK),
        tail_idx.reshape(total_chunks, GATHER_CHUNK),
        entity_table,
        relation_table,
    )
